# baseline (speedup 1.0000x reference)
import functools
import math

import jax
import jax.numpy as jnp
from jax import lax
from jax.experimental import pallas as pl
from jax.experimental.pallas import tpu as pltpu
from jax.experimental.pallas import tpu_sc as plsc

D = 64
T = 200
B = 1024
NC, NS = 2, 16
NW = NC * NS
N = B * T
RPW = N // NW
NB = 256
NSTEP = RPW // NB
NBUF = 5
PREF = 2
RIND = NB
PT = T + NB
SCALE = math.sqrt(D)


def _pos_body(out_ref):
    i = lax.broadcasted_iota(jnp.int32, (PT, D), 0)
    t = lax.rem(i, T).astype(jnp.float32)
    j = lax.broadcasted_iota(jnp.int32, (PT, D), 1)
    k2 = ((j >> 1) << 1).astype(jnp.float32)
    inv_freq = jnp.exp(k2 * (-math.log(10000.0) / D))
    ang = t * inv_freq
    out_ref[...] = jnp.where((j & 1) == 0, jnp.sin(ang), jnp.cos(ang))


def _pos_table():
    return pl.pallas_call(
        _pos_body,
        out_shape=jax.ShapeDtypeStruct((PT, D), jnp.float32),
    )()


_MESH = plsc.VectorSubcoreMesh(core_axis_name="c", subcore_axis_name="s")


@functools.partial(
    pl.kernel,
    out_type=jax.ShapeDtypeStruct((N, D), jnp.float32),
    mesh=_MESH,
    scratch_types=[
        pltpu.VMEM((RPW,), jnp.int32),
        *[pltpu.VMEM((NB, D), jnp.float32) for _ in range(NBUF)],
        pltpu.VMEM((PT * D,), jnp.float32),
        pltpu.SemaphoreType.DMA((NBUF,)),
        pltpu.SemaphoreType.DMA((NBUF,)),
    ],
    compiler_params=pltpu.CompilerParams(use_tc_tiling_on_sc=False),
)
def _embed(seq_hbm, table, pose_hbm, out,
           idx_all, r0, r1, r2, r3, r4, pos_v, semg, semw):
    rows = (r0, r1, r2, r3, r4)
    wid = lax.axis_index("s") * NC + lax.axis_index("c")
    base0 = wid * RPW
    pltpu.sync_copy(pose_hbm, pos_v)
    pltpu.sync_copy(seq_hbm.at[pl.ds(pl.multiple_of(base0, NB), RPW)], idx_all)

    def issue_fetches(g, k):
        base = g * NB
        rbuf = rows[k]
        for j in range(RIND // 16):
            idxv = idx_all[pl.ds(base + j * 16, 16)]
            pltpu.async_copy(
                table.at[idxv], rbuf.at[pl.ds(j * 16, 16)], semg.at[k])


    def wait_fetches(k):
        pltpu.make_async_copy(
            table.at[pl.ds(0, NB)], rows[k], semg.at[k]).wait()

    def start_wb(g, k):
        off = pl.multiple_of(base0 + g * NB, NB)
        pltpu.async_copy(rows[k], out.at[pl.ds(off, NB)], semw.at[k])

    def wait_wb(k):
        pltpu.make_async_copy(
            rows[k], out.at[pl.ds(0, NB)], semw.at[k]).wait()

    for k in range(PREF):
        issue_fetches(k, k)

    def outer(i, carry):
        for k in range(NBUF):
            g = i * NBUF + k
            kp = (k + PREF) % NBUF

            @pl.when(g + PREF >= NBUF)
            def _():
                wait_wb(kp)

            @pl.when(g + PREF < NSTEP)
            def _():
                issue_fetches(g + PREF, kp)

            wait_fetches(k)
            t0 = lax.rem(base0 + g * NB, T)
            rbuf = rows[k]

            @plsc.parallel_loop(0, NB, unroll=4)
            def _(r):
                poff = pl.multiple_of((t0 + r) * D, D)
                for c in range(D // 16):
                    sl = pl.ds(c * 16, 16)
                    rbuf[r, sl] = (rbuf[r, sl] * SCALE
                                   + pos_v[pl.ds(poff + c * 16, 16)])

            start_wb(g, k)
        return carry

    lax.fori_loop(0, NSTEP // NBUF, outer, 0)
    for g in range(NSTEP - (NBUF - PREF), NSTEP):
        wait_wb(g % NBUF)


def kernel(sequences, token_emb):
    seq_flat = sequences.reshape(N)
    pos_ext = _pos_table().reshape(PT * D)
    out = _embed(seq_flat, token_emb, pos_ext)
    return out.reshape(B, T, D)


# speedup vs baseline: 1.0001x; 1.0001x over previous
import functools
import math

import jax
import jax.numpy as jnp
from jax import lax
from jax.experimental import pallas as pl
from jax.experimental.pallas import tpu as pltpu
from jax.experimental.pallas import tpu_sc as plsc

D = 64
T = 200
B = 1024
NC, NS = 2, 16
NW = NC * NS
N = B * T
RPW = N // NW
NB = 256
NSTEP = RPW // NB
NBUF = 5
PREF = 2
RIND = NB
PT = T + NB
SCALE = math.sqrt(D)


def _pos_body(out_ref):
    i = lax.broadcasted_iota(jnp.int32, (PT, D), 0)
    t = lax.rem(i, T).astype(jnp.float32)
    j = lax.broadcasted_iota(jnp.int32, (PT, D), 1)
    k2 = ((j >> 1) << 1).astype(jnp.float32)
    inv_freq = jnp.exp(k2 * (-math.log(10000.0) / D))
    ang = t * inv_freq
    out_ref[...] = jnp.where((j & 1) == 0, jnp.sin(ang), jnp.cos(ang))


def _pos_table():
    return pl.pallas_call(
        _pos_body,
        out_shape=jax.ShapeDtypeStruct((PT, D), jnp.float32),
    )()


_MESH = plsc.VectorSubcoreMesh(core_axis_name="c", subcore_axis_name="s",
                               num_cores=2, num_subcores=16)


@functools.partial(
    pl.kernel,
    out_type=jax.ShapeDtypeStruct((N, D), jnp.float32),
    mesh=_MESH,
    scratch_types=[
        pltpu.VMEM((RPW,), jnp.int32),
        *[pltpu.VMEM((NB, D), jnp.float32) for _ in range(NBUF)],
        pltpu.VMEM((PT * D,), jnp.float32),
        pltpu.SemaphoreType.DMA((NBUF,)),
        pltpu.SemaphoreType.DMA((NBUF,)),
    ],
    compiler_params=pltpu.CompilerParams(use_tc_tiling_on_sc=False),
)
def _embed(seq_hbm, table, pose_hbm, out,
           idx_all, r0, r1, r2, r3, r4, pos_v, semg, semw):
    rows = (r0, r1, r2, r3, r4)
    wid = lax.axis_index("s") * NC + lax.axis_index("c")
    base0 = wid * RPW
    pltpu.sync_copy(pose_hbm, pos_v)
    pltpu.sync_copy(seq_hbm.at[pl.ds(pl.multiple_of(base0, NB), RPW)], idx_all)

    def issue_fetches(g, k):
        base = g * NB
        rbuf = rows[k]
        for j in range(RIND // 16):
            idxv = idx_all[pl.ds(base + j * 16, 16)]
            pltpu.async_copy(
                table.at[idxv], rbuf.at[pl.ds(j * 16, 16)], semg.at[k])


    def wait_fetches(k):
        pltpu.make_async_copy(
            table.at[pl.ds(0, NB)], rows[k], semg.at[k]).wait()

    def start_wb(g, k):
        off = pl.multiple_of(base0 + g * NB, NB)
        pltpu.async_copy(rows[k], out.at[pl.ds(off, NB)], semw.at[k])

    def wait_wb(k):
        pltpu.make_async_copy(
            rows[k], out.at[pl.ds(0, NB)], semw.at[k]).wait()

    for k in range(PREF):
        issue_fetches(k, k)

    def outer(i, carry):
        for k in range(NBUF):
            g = i * NBUF + k
            kp = (k + PREF) % NBUF

            @pl.when(g + PREF >= NBUF)
            def _():
                wait_wb(kp)

            @pl.when(g + PREF < NSTEP)
            def _():
                issue_fetches(g + PREF, kp)

            wait_fetches(k)
            t0 = lax.rem(base0 + g * NB, T)
            rbuf = rows[k]

            @plsc.parallel_loop(0, NB, unroll=4)
            def _(r):
                poff = pl.multiple_of((t0 + r) * D, D)
                for c in range(D // 16):
                    sl = pl.ds(c * 16, 16)
                    rbuf[r, sl] = (rbuf[r, sl] * SCALE
                                   + pos_v[pl.ds(poff + c * 16, 16)])

            start_wb(g, k)
        return carry

    lax.fori_loop(0, NSTEP // NBUF, outer, 0)
    for g in range(NSTEP - (NBUF - PREF), NSTEP):
        wait_wb(g % NBUF)


def kernel(sequences, token_emb):
    seq_flat = sequences.reshape(N)
    pos_ext = _pos_table().reshape(PT * D)
    out = _embed(seq_flat, token_emb, pos_ext)
    return out.reshape(B, T, D)


# speedup vs baseline: 1.0005x; 1.0004x over previous
import functools
import math

import jax
import jax.numpy as jnp
from jax import lax
from jax.experimental import pallas as pl
from jax.experimental.pallas import tpu as pltpu
from jax.experimental.pallas import tpu_sc as plsc

D = 64
T = 200
B = 1024
NC, NS = 2, 16
NW = NC * NS
N = B * T
RPW = N // NW
NB = 256
NSTEP = RPW // NB
NBUF = 5
PREF = 3
RIND = NB
PT = T + NB
SCALE = math.sqrt(D)


def _pos_body(out_ref):
    i = lax.broadcasted_iota(jnp.int32, (PT, D), 0)
    t = lax.rem(i, T).astype(jnp.float32)
    j = lax.broadcasted_iota(jnp.int32, (PT, D), 1)
    k2 = ((j >> 1) << 1).astype(jnp.float32)
    inv_freq = jnp.exp(k2 * (-math.log(10000.0) / D))
    ang = t * inv_freq
    out_ref[...] = jnp.where((j & 1) == 0, jnp.sin(ang), jnp.cos(ang))


def _pos_table():
    return pl.pallas_call(
        _pos_body,
        out_shape=jax.ShapeDtypeStruct((PT, D), jnp.float32),
    )()


_MESH = plsc.VectorSubcoreMesh(core_axis_name="c", subcore_axis_name="s",
                               num_cores=2, num_subcores=16)


@functools.partial(
    pl.kernel,
    out_type=jax.ShapeDtypeStruct((N, D), jnp.float32),
    mesh=_MESH,
    scratch_types=[
        pltpu.VMEM((RPW,), jnp.int32),
        *[pltpu.VMEM((NB, D), jnp.float32) for _ in range(NBUF)],
        pltpu.VMEM((PT * D,), jnp.float32),
        pltpu.SemaphoreType.DMA((NBUF,)),
        pltpu.SemaphoreType.DMA((NBUF,)),
    ],
    compiler_params=pltpu.CompilerParams(use_tc_tiling_on_sc=False),
)
def _embed(seq_hbm, table, pose_hbm, out,
           idx_all, r0, r1, r2, r3, r4, pos_v, semg, semw):
    rows = (r0, r1, r2, r3, r4)
    wid = lax.axis_index("s") * NC + lax.axis_index("c")
    base0 = wid * RPW
    pltpu.sync_copy(pose_hbm, pos_v)
    pltpu.sync_copy(seq_hbm.at[pl.ds(pl.multiple_of(base0, NB), RPW)], idx_all)

    def issue_fetches(g, k):
        base = g * NB
        rbuf = rows[k]
        for j in range(RIND // 16):
            idxv = idx_all[pl.ds(base + j * 16, 16)]
            pltpu.async_copy(
                table.at[idxv], rbuf.at[pl.ds(j * 16, 16)], semg.at[k])


    def wait_fetches(k):
        pltpu.make_async_copy(
            table.at[pl.ds(0, NB)], rows[k], semg.at[k]).wait()

    def start_wb(g, k):
        off = pl.multiple_of(base0 + g * NB, NB)
        pltpu.async_copy(rows[k], out.at[pl.ds(off, NB)], semw.at[k])

    def wait_wb(k):
        pltpu.make_async_copy(
            rows[k], out.at[pl.ds(0, NB)], semw.at[k]).wait()

    for k in range(PREF):
        issue_fetches(k, k)

    def outer(i, carry):
        for k in range(NBUF):
            g = i * NBUF + k
            kp = (k + PREF) % NBUF

            @pl.when(g + PREF >= NBUF)
            def _():
                wait_wb(kp)

            @pl.when(g + PREF < NSTEP)
            def _():
                issue_fetches(g + PREF, kp)

            wait_fetches(k)
            t0 = lax.rem(base0 + g * NB, T)
            rbuf = rows[k]

            @plsc.parallel_loop(0, NB, unroll=4)
            def _(r):
                poff = pl.multiple_of((t0 + r) * D, D)
                for c in range(D // 16):
                    sl = pl.ds(c * 16, 16)
                    rbuf[r, sl] = (rbuf[r, sl] * SCALE
                                   + pos_v[pl.ds(poff + c * 16, 16)])

            start_wb(g, k)
        return carry

    lax.fori_loop(0, NSTEP // NBUF, outer, 0)
    for g in range(NSTEP - (NBUF - PREF), NSTEP):
        wait_wb(g % NBUF)


def kernel(sequences, token_emb):
    seq_flat = sequences.reshape(N)
    pos_ext = _pos_table().reshape(PT * D)
    out = _embed(seq_flat, token_emb, pos_ext)
    return out.reshape(B, T, D)


# speedup vs baseline: 1.0018x; 1.0013x over previous
import functools
import math

import jax
import jax.numpy as jnp
from jax import lax
from jax.experimental import pallas as pl
from jax.experimental.pallas import tpu as pltpu
from jax.experimental.pallas import tpu_sc as plsc

D = 64
T = 200
B = 1024
NC, NS = 2, 16
NW = NC * NS
N = B * T
RPW = N // NW
NB = 128
NSTEP = RPW // NB
NBUF = 5
PREF = 2
RIND = NB
PT = T + NB
SCALE = math.sqrt(D)


def _pos_body(out_ref):
    i = lax.broadcasted_iota(jnp.int32, (PT, D), 0)
    t = lax.rem(i, T).astype(jnp.float32)
    j = lax.broadcasted_iota(jnp.int32, (PT, D), 1)
    k2 = ((j >> 1) << 1).astype(jnp.float32)
    inv_freq = jnp.exp(k2 * (-math.log(10000.0) / D))
    ang = t * inv_freq
    out_ref[...] = jnp.where((j & 1) == 0, jnp.sin(ang), jnp.cos(ang))


def _pos_table():
    return pl.pallas_call(
        _pos_body,
        out_shape=jax.ShapeDtypeStruct((PT, D), jnp.float32),
    )()


_MESH = plsc.VectorSubcoreMesh(core_axis_name="c", subcore_axis_name="s",
                               num_cores=2, num_subcores=16)


@functools.partial(
    pl.kernel,
    out_type=jax.ShapeDtypeStruct((N, D), jnp.float32),
    mesh=_MESH,
    scratch_types=[
        pltpu.VMEM((RPW,), jnp.int32),
        *[pltpu.VMEM((NB, D), jnp.float32) for _ in range(NBUF)],
        pltpu.VMEM((PT * D,), jnp.float32),
        pltpu.SemaphoreType.DMA((NBUF,)),
        pltpu.SemaphoreType.DMA((NBUF,)),
    ],
    compiler_params=pltpu.CompilerParams(use_tc_tiling_on_sc=False),
)
def _embed(seq_hbm, table, pose_hbm, out,
           idx_all, r0, r1, r2, r3, r4, pos_v, semg, semw):
    rows = (r0, r1, r2, r3, r4)
    wid = lax.axis_index("s") * NC + lax.axis_index("c")
    base0 = wid * RPW
    pltpu.sync_copy(pose_hbm, pos_v)
    pltpu.sync_copy(seq_hbm.at[pl.ds(pl.multiple_of(base0, NB), RPW)], idx_all)

    def issue_fetches(g, k):
        base = g * NB
        rbuf = rows[k]
        for j in range(RIND // 16):
            idxv = idx_all[pl.ds(base + j * 16, 16)]
            pltpu.async_copy(
                table.at[idxv], rbuf.at[pl.ds(j * 16, 16)], semg.at[k])


    def wait_fetches(k):
        pltpu.make_async_copy(
            table.at[pl.ds(0, NB)], rows[k], semg.at[k]).wait()

    def start_wb(g, k):
        off = pl.multiple_of(base0 + g * NB, NB)
        pltpu.async_copy(rows[k], out.at[pl.ds(off, NB)], semw.at[k])

    def wait_wb(k):
        pltpu.make_async_copy(
            rows[k], out.at[pl.ds(0, NB)], semw.at[k]).wait()

    for k in range(PREF):
        issue_fetches(k, k)

    def outer(i, carry):
        for k in range(NBUF):
            g = i * NBUF + k
            kp = (k + PREF) % NBUF

            @pl.when(g + PREF >= NBUF)
            def _():
                wait_wb(kp)

            @pl.when(g + PREF < NSTEP)
            def _():
                issue_fetches(g + PREF, kp)

            wait_fetches(k)
            t0 = lax.rem(base0 + g * NB, T)
            rbuf = rows[k]

            @plsc.parallel_loop(0, NB, unroll=4)
            def _(r):
                poff = pl.multiple_of((t0 + r) * D, D)
                for c in range(D // 16):
                    sl = pl.ds(c * 16, 16)
                    rbuf[r, sl] = (rbuf[r, sl] * SCALE
                                   + pos_v[pl.ds(poff + c * 16, 16)])

            start_wb(g, k)
        return carry

    lax.fori_loop(0, NSTEP // NBUF, outer, 0)
    for g in range(NSTEP - (NBUF - PREF), NSTEP):
        wait_wb(g % NBUF)


def kernel(sequences, token_emb):
    seq_flat = sequences.reshape(N)
    pos_ext = _pos_table().reshape(PT * D)
    out = _embed(seq_flat, token_emb, pos_ext)
    return out.reshape(B, T, D)
